# baseline (speedup 1.0000x reference)
import functools

import jax
import jax.numpy as jnp
from jax import lax
from jax.experimental import pallas as pl
from jax.experimental.pallas import tpu as pltpu
from jax.experimental.pallas import tpu_sc as plsc

N_FIELDS = 26
VOCAB = 100000
D = 16
BATCH = 16384
N_CONT = 13

NC = 2
NS = 16
NW = NC * NS
DISC_STRIPS = N_FIELDS * D
CONT_STRIPS = N_CONT * D
DSPW = DISC_STRIPS // NW
CSPW = -(-CONT_STRIPS // NW)
HALF = BATCH // 2


def _sc_body(tab_t, xd_t, xc_t, cwf, outh, row_v, idx_v, out_v, cw_v, rsem, wsem):
    wid = lax.axis_index("s") * NC + lax.axis_index("c")
    pltpu.sync_copy(cwf, cw_v)

    def wait_out_write():
        pltpu.make_async_copy(out_v, outh.at[0, 0, pl.ds(0, HALF)], wsem).wait()

    def cont_strip(ct):
        q = wid * CSPW + ct

        @pl.when(q < CONT_STRIPS)
        def _():
            j = jnp.right_shift(q, 4)
            d = jnp.bitwise_and(q, 15)
            cws = plsc.load_gather(cw_v, [jnp.broadcast_to(q, (16,))])
            for h in range(2):
                wait_out_write()
                pltpu.sync_copy(xc_t.at[j, pl.ds(h * HALF, HALF)], out_v)

                def ck(k, carry):
                    vs = [out_v[pl.ds(k * 128 + i * 16, 16)] for i in range(8)]
                    prods = [cws * v for v in vs]
                    for i in range(8):
                        out_v[pl.ds(k * 128 + i * 16, 16)] = prods[i]
                    return carry

                lax.fori_loop(0, HALF // 128, ck, 0, unroll=2)
                pltpu.async_copy(out_v,
                                 outh.at[N_FIELDS + j, d, pl.ds(h * HALF, HALF)],
                                 wsem)

    for t in range(DSPW):
        strip = wid * DSPW + t
        f = jnp.right_shift(strip, 4)
        d = jnp.bitwise_and(strip, 15)
        row_cp = pltpu.async_copy(tab_t.at[f, d], row_v, rsem)
        if t == 0:
            pltpu.sync_copy(xd_t.at[f], idx_v)
        else:
            prev_f = jnp.right_shift(strip - 1, 4)

            @pl.when(f != prev_f)
            def _():
                pltpu.sync_copy(xd_t.at[f], idx_v)

        if 1 <= t <= CSPW:
            cont_strip(t - 1)
        row_cp.wait()
        for h in range(2):
            if not (t == 0 and h == 0):
                wait_out_write()

            def gk(k, carry):
                ivs = [idx_v[pl.ds(h * HALF + k * 256 + i * 16, 16)]
                       for i in range(16)]
                vals = [plsc.load_gather(row_v, [iv]) for iv in ivs]
                for i in range(16):
                    out_v[pl.ds(k * 256 + i * 16, 16)] = vals[i]
                return carry

            lax.fori_loop(0, HALF // 256, gk, 0, unroll=4)
            pltpu.async_copy(out_v, outh.at[f, d, pl.ds(h * HALF, HALF)], wsem)

    wait_out_write()


_sc_call = pl.kernel(
    _sc_body,
    out_type=jax.ShapeDtypeStruct((N_FIELDS + N_CONT, D, BATCH), jnp.float32),
    mesh=plsc.VectorSubcoreMesh(core_axis_name="c", subcore_axis_name="s"),
    compiler_params=pltpu.CompilerParams(use_tc_tiling_on_sc=True,
                                         needs_layout_passes=False),
    scratch_types=[
        pltpu.VMEM((VOCAB,), jnp.float32),
        pltpu.VMEM((BATCH,), jnp.int32),
        pltpu.VMEM((HALF,), jnp.float32),
        pltpu.VMEM((CONT_STRIPS,), jnp.float32),
        pltpu.SemaphoreType.DMA,
        pltpu.SemaphoreType.DMA,
    ],
)


@jax.jit
def kernel(x_disc, x_cont, tables, cont_w):
    tab_t = tables.transpose(0, 2, 1)
    xd_t = x_disc.astype(jnp.int32).T
    xc_t = x_cont.T
    cwf = cont_w.reshape(CONT_STRIPS)
    out_t = _sc_call(tab_t, xd_t, xc_t, cwf)
    return out_t.transpose(0, 2, 1)
